# baseline (speedup 1.0000x reference)
import functools

import jax
import jax.numpy as jnp
from jax import lax
from jax.experimental import pallas as pl
from jax.experimental.pallas import tpu as pltpu
from jax.experimental.pallas import tpu_sc as plsc

N = 4096
N_PAIRS = 65536
F_DIM = 64
F_PAD = 128


def _mm_body(h_ref, w_ref, o_ref):
    o_ref[...] = jnp.dot(h_ref[...], w_ref[...],
                         preferred_element_type=jnp.float32)


def _proj(h, w, bm=1024):
    m, k = h.shape
    n = w.shape[1]
    return pl.pallas_call(
        _mm_body,
        grid=(m // bm,),
        in_specs=[pl.BlockSpec((bm, k), lambda i: (i, 0)),
                  pl.BlockSpec((k, n), lambda i: (0, 0))],
        out_specs=pl.BlockSpec((bm, n), lambda i: (i, 0)),
        out_shape=jax.ShapeDtypeStruct((m, n), jnp.float32),
    )(h, w)


def _agg0_body(s_ref, t_ref, self_ref, w_ref, o_ref):
    t = t_ref[...].astype(jnp.bfloat16)
    a0 = jnp.dot(s_ref[0].astype(jnp.bfloat16), t[:, :128],
                 preferred_element_type=jnp.float32)
    a1 = jnp.dot(s_ref[1].astype(jnp.bfloat16), t[:, 128:256],
                 preferred_element_type=jnp.float32)
    h1 = jnp.maximum(jnp.concatenate([a0, a1], axis=1) + self_ref[...], 0.0)
    o_ref[...] = jnp.dot(h1, w_ref[...], preferred_element_type=jnp.float32)


def _agg0(supports, proj_other, proj_self, wcat1, bm=256):
    return pl.pallas_call(
        _agg0_body,
        grid=(N // bm,),
        in_specs=[
            pl.BlockSpec((2, bm, N), lambda m: (0, m, 0)),
            pl.BlockSpec((N, 256), lambda m: (0, 0)),
            pl.BlockSpec((bm, 256), lambda m: (m, 1)),
            pl.BlockSpec((256, 256), lambda m: (0, 0)),
        ],
        out_specs=pl.BlockSpec((bm, 256), lambda m: (m, 0)),
        out_shape=jax.ShapeDtypeStruct((N, 256), jnp.float32),
    )(supports, proj_other, proj_self, wcat1)


def _agg1_body(s_ref, t_ref, self_ref, dw_ref, o_ref):
    t = t_ref[...].astype(jnp.bfloat16)
    a0 = jnp.dot(s_ref[0].astype(jnp.bfloat16), t[:, :64],
                 preferred_element_type=jnp.float32)
    a1 = jnp.dot(s_ref[1].astype(jnp.bfloat16), t[:, 64:128],
                 preferred_element_type=jnp.float32)
    h2 = jnp.maximum(jnp.concatenate([a0, a1], axis=1) + self_ref[...], 0.0)
    o_ref[...] = jnp.maximum(
        jnp.dot(h2, dw_ref[...], preferred_element_type=jnp.float32), 0.0)


def _agg1(supports, q_other, q_self, dense_w, bm=256):
    return pl.pallas_call(
        _agg1_body,
        grid=(N // bm,),
        in_specs=[
            pl.BlockSpec((2, bm, N), lambda m: (0, m, 0)),
            pl.BlockSpec((N, 128), lambda m: (0, 0)),
            pl.BlockSpec((bm, 128), lambda m: (m, 1)),
            pl.BlockSpec((128, F_PAD), lambda m: (0, 0)),
        ],
        out_specs=pl.BlockSpec((bm, F_PAD), lambda m: (m, 0)),
        out_shape=jax.ShapeDtypeStruct((N, F_PAD), jnp.float32),
    )(supports, q_other, q_self, dense_w)


def _fin_body(r_ref, p_ref, wr_ref, wc_ref, o_ref):
    rp = r_ref[:, :F_DIM] * p_ref[:, :F_DIM]
    m0 = wc_ref[0, 0] * wr_ref[0:1, :] + wc_ref[1, 0] * wr_ref[1:2, :]
    m1 = wc_ref[0, 1] * wr_ref[0:1, :] + wc_ref[1, 1] * wr_ref[1:2, :]
    b0 = jnp.sum(rp * m0, axis=1, keepdims=True)
    b1 = jnp.sum(rp * m1, axis=1, keepdims=True)
    o_ref[...] = jnp.maximum(jnp.concatenate([b0, b1], axis=1), 0.0)


def _finish(r_rows, p_rows, w_relation, weight_classifier, br=8192):
    return pl.pallas_call(
        _fin_body,
        grid=(N_PAIRS // br,),
        in_specs=[
            pl.BlockSpec((br, F_PAD), lambda i: (i, 0)),
            pl.BlockSpec((br, F_PAD), lambda i: (i, 0)),
            pl.BlockSpec((2, F_DIM), lambda i: (0, 0)),
            pl.BlockSpec(memory_space=pltpu.SMEM),
        ],
        out_specs=pl.BlockSpec((br, 2), lambda i: (i, 0)),
        out_shape=jax.ShapeDtypeStruct((N_PAIRS, 2), jnp.float32),
    )(r_rows, p_rows, w_relation, weight_classifier)


_NC, _NS = 2, 16
_NW = _NC * _NS
_CHUNK = 128
_CPW = N_PAIRS // _NW // _CHUNK
_PPW = N_PAIRS // _NW


def _sc_gather(f_r, f_p, idx_r2, idx_p2):
    mesh = plsc.VectorSubcoreMesh(core_axis_name="c", subcore_axis_name="s",
                                  num_cores=_NC, num_subcores=_NS)

    @functools.partial(
        pl.kernel,
        out_type=(jax.ShapeDtypeStruct((N_PAIRS, F_PAD), jnp.float32),
                  jax.ShapeDtypeStruct((N_PAIRS, F_PAD), jnp.float32)),
        mesh=mesh,
        scratch_types=[
            pltpu.VMEM((_CPW, _CHUNK), jnp.int32),
            pltpu.VMEM((_CPW, _CHUNK), jnp.int32),
            pltpu.VMEM((_CHUNK, F_PAD), jnp.float32),
            pltpu.VMEM((_CHUNK, F_PAD), jnp.float32),
            pltpu.SemaphoreType.DMA,
            pltpu.SemaphoreType.DMA,
        ],
    )
    def k(fr_hbm, fp_hbm, ir_hbm, ip_hbm, or_hbm, op_hbm,
          ir_v, ip_v, br_v, bp_v, sr, sp):
        wid = lax.axis_index("s") * _NC + lax.axis_index("c")
        blk = wid * _CPW
        pltpu.sync_copy(ir_hbm.at[pl.ds(blk, _CPW)], ir_v)
        pltpu.sync_copy(ip_hbm.at[pl.ds(blk, _CPW)], ip_v)
        for c in range(_CPW):
            cr = pltpu.async_copy(fr_hbm.at[ir_v.at[c]], br_v, sr)
            cp = pltpu.async_copy(fp_hbm.at[ip_v.at[c]], bp_v, sp)
            out_off = wid * _PPW + c * _CHUNK
            cr.wait()
            pltpu.sync_copy(br_v, or_hbm.at[pl.ds(out_off, _CHUNK)])
            cp.wait()
            pltpu.sync_copy(bp_v, op_hbm.at[pl.ds(out_off, _CHUNK)])

    return k(f_r, f_p, idx_r2, idx_p2)


def kernel(RNA_supports, protein_supports, RNA_inputs, protein_inputs,
           enc_w0, enc_w1, self_w0, self_w1,
           dense_w_rna, dense_w_prot, w_relation, weight_classifier,
           RNA_indices, protein_indices):
    wcat0 = jnp.concatenate([enc_w0[0], enc_w0[1], self_w0], axis=1)
    wcat1 = jnp.concatenate([enc_w1[0], enc_w1[1], self_w1], axis=1)

    pr = _proj(RNA_inputs, wcat0)
    pp = _proj(protein_inputs, wcat0)

    q_r = _agg0(RNA_supports, pp, pr, wcat1)
    q_p = _agg0(protein_supports, pr, pp, wcat1)

    dwr = jnp.pad(dense_w_rna, ((0, 0), (0, F_PAD - F_DIM)))
    dwp = jnp.pad(dense_w_prot, ((0, 0), (0, F_PAD - F_DIM)))
    f_r = _agg1(RNA_supports, q_p, q_r, dwr)
    f_p = _agg1(protein_supports, q_r, q_p, dwp)

    r_rows, p_rows = _sc_gather(
        f_r, f_p,
        RNA_indices.reshape(_NW * _CPW, _CHUNK),
        protein_indices.reshape(_NW * _CPW, _CHUNK))
    return _finish(r_rows, p_rows, w_relation, weight_classifier)


# speedup vs baseline: 1.0047x; 1.0047x over previous
import functools

import jax
import jax.numpy as jnp
from jax import lax
from jax.experimental import pallas as pl
from jax.experimental.pallas import tpu as pltpu
from jax.experimental.pallas import tpu_sc as plsc

N = 4096
N_PAIRS = 65536
F_DIM = 64
F_PAD = 128


def _mm_body(h_ref, w_ref, o_ref):
    o_ref[...] = jnp.dot(h_ref[...], w_ref[...],
                         preferred_element_type=jnp.float32)


def _proj(h, w, bm=1024):
    m, k = h.shape
    n = w.shape[1]
    return pl.pallas_call(
        _mm_body,
        grid=(m // bm,),
        in_specs=[pl.BlockSpec((bm, k), lambda i: (i, 0)),
                  pl.BlockSpec((k, n), lambda i: (0, 0))],
        out_specs=pl.BlockSpec((bm, n), lambda i: (i, 0)),
        out_shape=jax.ShapeDtypeStruct((m, n), jnp.float32),
    )(h, w)


def _agg0_body(s_ref, t_ref, self_ref, w_ref, o_ref):
    t = t_ref[...].astype(jnp.bfloat16)
    a0 = jnp.dot(s_ref[0].astype(jnp.bfloat16), t[:, :128],
                 preferred_element_type=jnp.float32)
    a1 = jnp.dot(s_ref[1].astype(jnp.bfloat16), t[:, 128:256],
                 preferred_element_type=jnp.float32)
    h1 = jnp.maximum(jnp.concatenate([a0, a1], axis=1) + self_ref[...], 0.0)
    o_ref[...] = jnp.dot(h1, w_ref[...], preferred_element_type=jnp.float32)


def _agg0(supports, proj_other, proj_self, wcat1, bm=512):
    return pl.pallas_call(
        _agg0_body,
        grid=(N // bm,),
        in_specs=[
            pl.BlockSpec((2, bm, N), lambda m: (0, m, 0)),
            pl.BlockSpec((N, 256), lambda m: (0, 0)),
            pl.BlockSpec((bm, 256), lambda m: (m, 1)),
            pl.BlockSpec((256, 256), lambda m: (0, 0)),
        ],
        out_specs=pl.BlockSpec((bm, 256), lambda m: (m, 0)),
        out_shape=jax.ShapeDtypeStruct((N, 256), jnp.float32),
    )(supports, proj_other, proj_self, wcat1)


def _agg1_body(s_ref, t_ref, self_ref, dw_ref, o_ref):
    t = t_ref[...].astype(jnp.bfloat16)
    a0 = jnp.dot(s_ref[0].astype(jnp.bfloat16), t[:, :64],
                 preferred_element_type=jnp.float32)
    a1 = jnp.dot(s_ref[1].astype(jnp.bfloat16), t[:, 64:128],
                 preferred_element_type=jnp.float32)
    h2 = jnp.maximum(jnp.concatenate([a0, a1], axis=1) + self_ref[...], 0.0)
    o_ref[...] = jnp.maximum(
        jnp.dot(h2, dw_ref[...], preferred_element_type=jnp.float32), 0.0)


def _agg1(supports, q_other, q_self, dense_w, bm=512):
    return pl.pallas_call(
        _agg1_body,
        grid=(N // bm,),
        in_specs=[
            pl.BlockSpec((2, bm, N), lambda m: (0, m, 0)),
            pl.BlockSpec((N, 128), lambda m: (0, 0)),
            pl.BlockSpec((bm, 128), lambda m: (m, 1)),
            pl.BlockSpec((128, F_PAD), lambda m: (0, 0)),
        ],
        out_specs=pl.BlockSpec((bm, F_PAD), lambda m: (m, 0)),
        out_shape=jax.ShapeDtypeStruct((N, F_PAD), jnp.float32),
    )(supports, q_other, q_self, dense_w)


def _fin_body(r_ref, p_ref, wr_ref, wc_ref, o_ref):
    rp = r_ref[:, :F_DIM] * p_ref[:, :F_DIM]
    m0 = wc_ref[0, 0] * wr_ref[0:1, :] + wc_ref[1, 0] * wr_ref[1:2, :]
    m1 = wc_ref[0, 1] * wr_ref[0:1, :] + wc_ref[1, 1] * wr_ref[1:2, :]
    b0 = jnp.sum(rp * m0, axis=1, keepdims=True)
    b1 = jnp.sum(rp * m1, axis=1, keepdims=True)
    o_ref[...] = jnp.maximum(jnp.concatenate([b0, b1], axis=1), 0.0)


def _finish(r_rows, p_rows, w_relation, weight_classifier, br=8192):
    return pl.pallas_call(
        _fin_body,
        grid=(N_PAIRS // br,),
        in_specs=[
            pl.BlockSpec((br, F_PAD), lambda i: (i, 0)),
            pl.BlockSpec((br, F_PAD), lambda i: (i, 0)),
            pl.BlockSpec((2, F_DIM), lambda i: (0, 0)),
            pl.BlockSpec(memory_space=pltpu.SMEM),
        ],
        out_specs=pl.BlockSpec((br, 2), lambda i: (i, 0)),
        out_shape=jax.ShapeDtypeStruct((N_PAIRS, 2), jnp.float32),
    )(r_rows, p_rows, w_relation, weight_classifier)


_NC, _NS = 2, 16
_NW = _NC * _NS
_CHUNK = 128
_CPW = N_PAIRS // _NW // _CHUNK
_PPW = N_PAIRS // _NW


def _sc_gather_one(table, idx2):
    mesh = plsc.VectorSubcoreMesh(core_axis_name="c", subcore_axis_name="s",
                                  num_cores=_NC, num_subcores=_NS)

    @functools.partial(
        pl.kernel,
        out_type=jax.ShapeDtypeStruct((N_PAIRS, F_PAD), jnp.float32),
        mesh=mesh,
        scratch_types=[
            pltpu.VMEM((_CPW, _CHUNK), jnp.int32),
            pltpu.VMEM((_CHUNK, F_PAD), jnp.float32),
            pltpu.VMEM((_CHUNK, F_PAD), jnp.float32),
            pltpu.SemaphoreType.DMA,
            pltpu.SemaphoreType.DMA,
        ],
    )
    def k(t_hbm, i_hbm, o_hbm, i_v, b0_v, b1_v, s0, s1):
        wid = lax.axis_index("s") * _NC + lax.axis_index("c")
        pltpu.sync_copy(i_hbm.at[pl.ds(wid * _CPW, _CPW)], i_v)
        bufs = (b0_v, b1_v)
        sems = (s0, s1)
        copies = [None, None]
        for c in range(_CPW):
            copies[c % 2] = pltpu.async_copy(
                t_hbm.at[i_v.at[c]], bufs[c % 2], sems[c % 2])
            if c > 0:
                copies[(c - 1) % 2].wait()
                off = wid * _PPW + (c - 1) * _CHUNK
                pltpu.sync_copy(bufs[(c - 1) % 2], o_hbm.at[pl.ds(off, _CHUNK)])
        copies[(_CPW - 1) % 2].wait()
        off = wid * _PPW + (_CPW - 1) * _CHUNK
        pltpu.sync_copy(bufs[(_CPW - 1) % 2], o_hbm.at[pl.ds(off, _CHUNK)])

    return k(table, idx2)


def kernel(RNA_supports, protein_supports, RNA_inputs, protein_inputs,
           enc_w0, enc_w1, self_w0, self_w1,
           dense_w_rna, dense_w_prot, w_relation, weight_classifier,
           RNA_indices, protein_indices):
    wcat0 = jnp.concatenate([enc_w0[0], enc_w0[1], self_w0], axis=1)
    wcat1 = jnp.concatenate([enc_w1[0], enc_w1[1], self_w1], axis=1)

    pr = _proj(RNA_inputs, wcat0)
    pp = _proj(protein_inputs, wcat0)

    q_r = _agg0(RNA_supports, pp, pr, wcat1)
    q_p = _agg0(protein_supports, pr, pp, wcat1)

    dwr = jnp.pad(dense_w_rna, ((0, 0), (0, F_PAD - F_DIM)))
    dwp = jnp.pad(dense_w_prot, ((0, 0), (0, F_PAD - F_DIM)))
    f_r = _agg1(RNA_supports, q_p, q_r, dwr)
    r_rows = _sc_gather_one(f_r, RNA_indices.reshape(_NW * _CPW, _CHUNK))
    f_p = _agg1(protein_supports, q_r, q_p, dwp)
    p_rows = _sc_gather_one(f_p, protein_indices.reshape(_NW * _CPW, _CHUNK))
    return _finish(r_rows, p_rows, w_relation, weight_classifier)


# speedup vs baseline: 1.0139x; 1.0091x over previous
import functools

import jax
import jax.numpy as jnp
from jax import lax
from jax.experimental import pallas as pl
from jax.experimental.pallas import tpu as pltpu
from jax.experimental.pallas import tpu_sc as plsc

N = 4096
N_PAIRS = 65536
F_DIM = 64
F_PAD = 128


def _mm_body(h_ref, w_ref, o_ref):
    o_ref[...] = jnp.dot(h_ref[...], w_ref[...],
                         preferred_element_type=jnp.float32)


def _proj(h, w, bm=1024):
    m, k = h.shape
    n = w.shape[1]
    return pl.pallas_call(
        _mm_body,
        grid=(m // bm,),
        in_specs=[pl.BlockSpec((bm, k), lambda i: (i, 0)),
                  pl.BlockSpec((k, n), lambda i: (0, 0))],
        out_specs=pl.BlockSpec((bm, n), lambda i: (i, 0)),
        out_shape=jax.ShapeDtypeStruct((m, n), jnp.float32),
    )(h, w)


def _agg0_body(s_ref, t_ref, self_ref, w_ref, o_ref):
    t = t_ref[...].astype(jnp.bfloat16)
    a0 = jnp.dot(s_ref[0].astype(jnp.bfloat16), t[:, :128],
                 preferred_element_type=jnp.float32)
    a1 = jnp.dot(s_ref[1].astype(jnp.bfloat16), t[:, 128:256],
                 preferred_element_type=jnp.float32)
    h1 = jnp.maximum(jnp.concatenate([a0, a1], axis=1) + self_ref[...], 0.0)
    o_ref[...] = jnp.dot(h1, w_ref[...], preferred_element_type=jnp.float32)


def _agg0(supports, proj_other, proj_self, wcat1, bm=512):
    return pl.pallas_call(
        _agg0_body,
        grid=(N // bm,),
        in_specs=[
            pl.BlockSpec((2, bm, N), lambda m: (0, m, 0)),
            pl.BlockSpec((N, 256), lambda m: (0, 0)),
            pl.BlockSpec((bm, 256), lambda m: (m, 1)),
            pl.BlockSpec((256, 256), lambda m: (0, 0)),
        ],
        out_specs=pl.BlockSpec((bm, 256), lambda m: (m, 0)),
        out_shape=jax.ShapeDtypeStruct((N, 256), jnp.float32),
    )(supports, proj_other, proj_self, wcat1)


def _agg1_body(s_ref, t_ref, self_ref, dw_ref, o_ref):
    t = t_ref[...].astype(jnp.bfloat16)
    a0 = jnp.dot(s_ref[0].astype(jnp.bfloat16), t[:, :64],
                 preferred_element_type=jnp.float32)
    a1 = jnp.dot(s_ref[1].astype(jnp.bfloat16), t[:, 64:128],
                 preferred_element_type=jnp.float32)
    h2 = jnp.maximum(jnp.concatenate([a0, a1], axis=1) + self_ref[...], 0.0)
    o_ref[...] = jnp.maximum(
        jnp.dot(h2, dw_ref[...], preferred_element_type=jnp.float32), 0.0)


def _agg1(supports, q_other, q_self, dense_w, bm=512):
    return pl.pallas_call(
        _agg1_body,
        grid=(N // bm,),
        in_specs=[
            pl.BlockSpec((2, bm, N), lambda m: (0, m, 0)),
            pl.BlockSpec((N, 128), lambda m: (0, 0)),
            pl.BlockSpec((bm, 128), lambda m: (m, 1)),
            pl.BlockSpec((128, F_PAD), lambda m: (0, 0)),
        ],
        out_specs=pl.BlockSpec((bm, F_PAD), lambda m: (m, 0)),
        out_shape=jax.ShapeDtypeStruct((N, F_PAD), jnp.float32),
    )(supports, q_other, q_self, dense_w)


def _fin_body(r_ref, p_ref, wr_ref, wc_ref, o_ref):
    rp = r_ref[:, :F_DIM] * p_ref[:, :F_DIM]
    m0 = wc_ref[0, 0] * wr_ref[0:1, :] + wc_ref[1, 0] * wr_ref[1:2, :]
    m1 = wc_ref[0, 1] * wr_ref[0:1, :] + wc_ref[1, 1] * wr_ref[1:2, :]
    m = jnp.concatenate([m0, m1], axis=0)
    basis = lax.dot_general(rp, m, (((1,), (1,)), ((), ())),
                            preferred_element_type=jnp.float32)
    o_ref[...] = jnp.maximum(basis, 0.0)


def _finish(r_rows, p_rows, w_relation, weight_classifier, br=8192):
    return pl.pallas_call(
        _fin_body,
        grid=(N_PAIRS // br,),
        in_specs=[
            pl.BlockSpec((br, F_PAD), lambda i: (i, 0)),
            pl.BlockSpec((br, F_PAD), lambda i: (i, 0)),
            pl.BlockSpec((2, F_DIM), lambda i: (0, 0)),
            pl.BlockSpec(memory_space=pltpu.SMEM),
        ],
        out_specs=pl.BlockSpec((br, 2), lambda i: (i, 0)),
        out_shape=jax.ShapeDtypeStruct((N_PAIRS, 2), jnp.float32),
    )(r_rows, p_rows, w_relation, weight_classifier)


_NC, _NS = 2, 16
_NW = _NC * _NS
_CHUNK = 128
_CPW = N_PAIRS // _NW // _CHUNK
_PPW = N_PAIRS // _NW


def _sc_gather_one(table, idx2):
    mesh = plsc.VectorSubcoreMesh(core_axis_name="c", subcore_axis_name="s",
                                  num_cores=_NC, num_subcores=_NS)

    @functools.partial(
        pl.kernel,
        out_type=jax.ShapeDtypeStruct((N_PAIRS, F_PAD), jnp.float32),
        mesh=mesh,
        scratch_types=[
            pltpu.VMEM((_CPW, _CHUNK), jnp.int32),
            pltpu.VMEM((_CHUNK, F_PAD), jnp.float32),
            pltpu.VMEM((_CHUNK, F_PAD), jnp.float32),
            pltpu.SemaphoreType.DMA,
            pltpu.SemaphoreType.DMA,
        ],
    )
    def k(t_hbm, i_hbm, o_hbm, i_v, b0_v, b1_v, s0, s1):
        wid = lax.axis_index("s") * _NC + lax.axis_index("c")
        pltpu.sync_copy(i_hbm.at[pl.ds(wid * _CPW, _CPW)], i_v)
        bufs = (b0_v, b1_v)
        sems = (s0, s1)
        copies = [None, None]
        for c in range(_CPW):
            copies[c % 2] = pltpu.async_copy(
                t_hbm.at[i_v.at[c]], bufs[c % 2], sems[c % 2])
            if c > 0:
                copies[(c - 1) % 2].wait()
                off = wid * _PPW + (c - 1) * _CHUNK
                pltpu.sync_copy(bufs[(c - 1) % 2], o_hbm.at[pl.ds(off, _CHUNK)])
        copies[(_CPW - 1) % 2].wait()
        off = wid * _PPW + (_CPW - 1) * _CHUNK
        pltpu.sync_copy(bufs[(_CPW - 1) % 2], o_hbm.at[pl.ds(off, _CHUNK)])

    return k(table, idx2)


def kernel(RNA_supports, protein_supports, RNA_inputs, protein_inputs,
           enc_w0, enc_w1, self_w0, self_w1,
           dense_w_rna, dense_w_prot, w_relation, weight_classifier,
           RNA_indices, protein_indices):
    wcat0 = jnp.concatenate([enc_w0[0], enc_w0[1], self_w0], axis=1)
    wcat1 = jnp.concatenate([enc_w1[0], enc_w1[1], self_w1], axis=1)

    pr = _proj(RNA_inputs, wcat0)
    pp = _proj(protein_inputs, wcat0)

    q_r = _agg0(RNA_supports, pp, pr, wcat1)
    q_p = _agg0(protein_supports, pr, pp, wcat1)

    dwr = jnp.pad(dense_w_rna, ((0, 0), (0, F_PAD - F_DIM)))
    dwp = jnp.pad(dense_w_prot, ((0, 0), (0, F_PAD - F_DIM)))
    f_r = _agg1(RNA_supports, q_p, q_r, dwr)
    r_rows = _sc_gather_one(f_r, RNA_indices.reshape(_NW * _CPW, _CHUNK))
    f_p = _agg1(protein_supports, q_r, q_p, dwp)
    p_rows = _sc_gather_one(f_p, protein_indices.reshape(_NW * _CPW, _CHUNK))
    return _finish(r_rows, p_rows, w_relation, weight_classifier)


# speedup vs baseline: 1.0762x; 1.0615x over previous
import functools

import jax
import jax.numpy as jnp
from jax import lax
from jax.experimental import pallas as pl
from jax.experimental.pallas import tpu as pltpu
from jax.experimental.pallas import tpu_sc as plsc

N = 4096
N_PAIRS = 65536
F_DIM = 64
F_PAD = 128


def _mm_body(h_ref, w_ref, o_ref):
    o_ref[...] = jnp.dot(h_ref[...], w_ref[...],
                         preferred_element_type=jnp.float32)


def _proj(h, w, bm=1024):
    m, k = h.shape
    n = w.shape[1]
    return pl.pallas_call(
        _mm_body,
        grid=(m // bm,),
        in_specs=[pl.BlockSpec((bm, k), lambda i: (i, 0)),
                  pl.BlockSpec((k, n), lambda i: (0, 0))],
        out_specs=pl.BlockSpec((bm, n), lambda i: (i, 0)),
        out_shape=jax.ShapeDtypeStruct((m, n), jnp.float32),
    )(h, w)


def _agg0_body(s_ref, t_ref, self_ref, w_ref, o_ref):
    t = t_ref[...].astype(jnp.bfloat16)
    a0 = jnp.dot(s_ref[0].astype(jnp.bfloat16), t[:, :128],
                 preferred_element_type=jnp.float32)
    a1 = jnp.dot(s_ref[1].astype(jnp.bfloat16), t[:, 128:256],
                 preferred_element_type=jnp.float32)
    h1 = jnp.maximum(jnp.concatenate([a0, a1], axis=1) + self_ref[...], 0.0)
    o_ref[...] = jnp.dot(h1, w_ref[...], preferred_element_type=jnp.float32)


def _agg0(supports, proj_other, proj_self, wcat1, bm=512):
    return pl.pallas_call(
        _agg0_body,
        grid=(N // bm,),
        in_specs=[
            pl.BlockSpec((2, bm, N), lambda m: (0, m, 0)),
            pl.BlockSpec((N, 256), lambda m: (0, 0)),
            pl.BlockSpec((bm, 256), lambda m: (m, 1)),
            pl.BlockSpec((256, 256), lambda m: (0, 0)),
        ],
        out_specs=pl.BlockSpec((bm, 256), lambda m: (m, 0)),
        out_shape=jax.ShapeDtypeStruct((N, 256), jnp.float32),
    )(supports, proj_other, proj_self, wcat1)


def _agg1_body(s_ref, t_ref, self_ref, dw_ref, o_ref):
    t = t_ref[...].astype(jnp.bfloat16)
    a0 = jnp.dot(s_ref[0].astype(jnp.bfloat16), t[:, :64],
                 preferred_element_type=jnp.float32)
    a1 = jnp.dot(s_ref[1].astype(jnp.bfloat16), t[:, 64:128],
                 preferred_element_type=jnp.float32)
    h2 = jnp.maximum(jnp.concatenate([a0, a1], axis=1) + self_ref[...], 0.0)
    o_ref[...] = jnp.maximum(
        jnp.dot(h2, dw_ref[...], preferred_element_type=jnp.float32), 0.0)


def _agg1(supports, q_other, q_self, dense_w, bm=512):
    return pl.pallas_call(
        _agg1_body,
        grid=(N // bm,),
        in_specs=[
            pl.BlockSpec((2, bm, N), lambda m: (0, m, 0)),
            pl.BlockSpec((N, 128), lambda m: (0, 0)),
            pl.BlockSpec((bm, 128), lambda m: (m, 1)),
            pl.BlockSpec((128, F_PAD), lambda m: (0, 0)),
        ],
        out_specs=pl.BlockSpec((bm, F_PAD), lambda m: (m, 0)),
        out_shape=jax.ShapeDtypeStruct((N, F_PAD), jnp.float32),
    )(supports, q_other, q_self, dense_w)


def _fin_body(rp_ref, wr_ref, wc_ref, o_ref):
    rp = rp_ref[:, :F_DIM]
    m0 = wc_ref[0, 0] * wr_ref[0:1, :] + wc_ref[1, 0] * wr_ref[1:2, :]
    m1 = wc_ref[0, 1] * wr_ref[0:1, :] + wc_ref[1, 1] * wr_ref[1:2, :]
    m = jnp.concatenate([m0, m1], axis=0)
    basis = lax.dot_general(rp, m, (((1,), (1,)), ((), ())),
                            preferred_element_type=jnp.float32)
    o_ref[...] = jnp.maximum(basis, 0.0)


def _finish(rp_rows, w_relation, weight_classifier, br=8192):
    return pl.pallas_call(
        _fin_body,
        grid=(N_PAIRS // br,),
        in_specs=[
            pl.BlockSpec((br, F_PAD), lambda i: (i, 0)),
            pl.BlockSpec((2, F_DIM), lambda i: (0, 0)),
            pl.BlockSpec(memory_space=pltpu.SMEM),
        ],
        out_specs=pl.BlockSpec((br, 2), lambda i: (i, 0)),
        out_shape=jax.ShapeDtypeStruct((N_PAIRS, 2), jnp.float32),
    )(rp_rows, w_relation, weight_classifier)


_NC, _NS = 2, 16
_NW = _NC * _NS
_CHUNK = 128
_CPW = N_PAIRS // _NW // _CHUNK
_PPW = N_PAIRS // _NW


def _sc_gather_mul(f_r, f_p, idx_r2, idx_p2):
    mesh = plsc.VectorSubcoreMesh(core_axis_name="c", subcore_axis_name="s",
                                  num_cores=_NC, num_subcores=_NS)

    @functools.partial(
        pl.kernel,
        out_type=jax.ShapeDtypeStruct((N_PAIRS, F_PAD), jnp.float32),
        mesh=mesh,
        scratch_types=[
            pltpu.VMEM((_CPW, _CHUNK), jnp.int32),
            pltpu.VMEM((_CPW, _CHUNK), jnp.int32),
            pltpu.VMEM((_CHUNK, F_PAD), jnp.float32),
            pltpu.VMEM((_CHUNK, F_PAD), jnp.float32),
            pltpu.VMEM((_CHUNK, F_PAD), jnp.float32),
            pltpu.VMEM((_CHUNK, F_PAD), jnp.float32),
            pltpu.VMEM((_CHUNK, F_PAD), jnp.float32),
            pltpu.SemaphoreType.DMA,
            pltpu.SemaphoreType.DMA,
            pltpu.SemaphoreType.DMA,
            pltpu.SemaphoreType.DMA,
        ],
    )
    def k(fr_hbm, fp_hbm, ir_hbm, ip_hbm, o_hbm,
          ir_v, ip_v, br0_v, br1_v, bp0_v, bp1_v, rp_v,
          sr0, sr1, sp0, sp1):
        wid = lax.axis_index("s") * _NC + lax.axis_index("c")
        pltpu.sync_copy(ir_hbm.at[pl.ds(wid * _CPW, _CPW)], ir_v)
        pltpu.sync_copy(ip_hbm.at[pl.ds(wid * _CPW, _CPW)], ip_v)
        rbufs, pbufs = (br0_v, br1_v), (bp0_v, bp1_v)
        rsems, psems = (sr0, sr1), (sp0, sp1)
        rcp = [None, None]
        pcp = [None, None]

        def _consume(slot, c):
            rcp[slot].wait()
            pcp[slot].wait()
            br, bp = rbufs[slot], pbufs[slot]

            def body(row, _):
                for g in range(F_PAD // 16):
                    sl = pl.ds(g * 16, 16)
                    rp_v[row, sl] = br[row, sl] * bp[row, sl]
                return _

            lax.fori_loop(0, _CHUNK, body, 0)
            off = wid * _PPW + c * _CHUNK
            pltpu.sync_copy(rp_v, o_hbm.at[pl.ds(off, _CHUNK)])

        for c in range(_CPW):
            slot = c % 2
            rcp[slot] = pltpu.async_copy(fr_hbm.at[ir_v.at[c]], rbufs[slot],
                                         rsems[slot])
            pcp[slot] = pltpu.async_copy(fp_hbm.at[ip_v.at[c]], pbufs[slot],
                                         psems[slot])
            if c > 0:
                _consume((c - 1) % 2, c - 1)
        _consume((_CPW - 1) % 2, _CPW - 1)

    return k(f_r, f_p, idx_r2, idx_p2)


def kernel(RNA_supports, protein_supports, RNA_inputs, protein_inputs,
           enc_w0, enc_w1, self_w0, self_w1,
           dense_w_rna, dense_w_prot, w_relation, weight_classifier,
           RNA_indices, protein_indices):
    wcat0 = jnp.concatenate([enc_w0[0], enc_w0[1], self_w0], axis=1)
    wcat1 = jnp.concatenate([enc_w1[0], enc_w1[1], self_w1], axis=1)

    pr = _proj(RNA_inputs, wcat0)
    pp = _proj(protein_inputs, wcat0)

    q_r = _agg0(RNA_supports, pp, pr, wcat1)
    q_p = _agg0(protein_supports, pr, pp, wcat1)

    dwr = jnp.pad(dense_w_rna, ((0, 0), (0, F_PAD - F_DIM)))
    dwp = jnp.pad(dense_w_prot, ((0, 0), (0, F_PAD - F_DIM)))
    f_r = _agg1(RNA_supports, q_p, q_r, dwr)
    f_p = _agg1(protein_supports, q_r, q_p, dwp)

    rp_rows = _sc_gather_mul(
        f_r, f_p,
        RNA_indices.reshape(_NW * _CPW, _CHUNK),
        protein_indices.reshape(_NW * _CPW, _CHUNK))
    return _finish(rp_rows, w_relation, weight_classifier)


# speedup vs baseline: 1.1792x; 1.0957x over previous
import functools

import jax
import jax.numpy as jnp
from jax import lax
from jax.experimental import pallas as pl
from jax.experimental.pallas import tpu as pltpu
from jax.experimental.pallas import tpu_sc as plsc

N = 4096
N_PAIRS = 65536
F_DIM = 64
F_PAD = 128


def _mm_body(h_ref, w_ref, o_ref):
    o_ref[...] = jnp.dot(h_ref[...], w_ref[...],
                         preferred_element_type=jnp.float32)


def _proj(h, w, bm=1024):
    m, k = h.shape
    n = w.shape[1]
    return pl.pallas_call(
        _mm_body,
        grid=(m // bm,),
        in_specs=[pl.BlockSpec((bm, k), lambda i: (i, 0)),
                  pl.BlockSpec((k, n), lambda i: (0, 0))],
        out_specs=pl.BlockSpec((bm, n), lambda i: (i, 0)),
        out_shape=jax.ShapeDtypeStruct((m, n), jnp.float32),
    )(h, w)


def _agg0_body(s_ref, t_ref, self_ref, w_ref, o_ref):
    t = t_ref[...].astype(jnp.bfloat16)
    a0 = jnp.dot(s_ref[0].astype(jnp.bfloat16), t[:, :128],
                 preferred_element_type=jnp.float32)
    a1 = jnp.dot(s_ref[1].astype(jnp.bfloat16), t[:, 128:256],
                 preferred_element_type=jnp.float32)
    h1 = jnp.maximum(jnp.concatenate([a0, a1], axis=1) + self_ref[...], 0.0)
    o_ref[...] = jnp.dot(h1, w_ref[...], preferred_element_type=jnp.float32)


def _agg0(supports, proj_other, proj_self, wcat1, bm=512):
    return pl.pallas_call(
        _agg0_body,
        grid=(N // bm,),
        in_specs=[
            pl.BlockSpec((2, bm, N), lambda m: (0, m, 0)),
            pl.BlockSpec((N, 256), lambda m: (0, 0)),
            pl.BlockSpec((bm, 256), lambda m: (m, 1)),
            pl.BlockSpec((256, 256), lambda m: (0, 0)),
        ],
        out_specs=pl.BlockSpec((bm, 256), lambda m: (m, 0)),
        out_shape=jax.ShapeDtypeStruct((N, 256), jnp.float32),
    )(supports, proj_other, proj_self, wcat1)


def _agg1_body(s_ref, t_ref, self_ref, dw_ref, o_ref):
    t = t_ref[...].astype(jnp.bfloat16)
    a0 = jnp.dot(s_ref[0].astype(jnp.bfloat16), t[:, :64],
                 preferred_element_type=jnp.float32)
    a1 = jnp.dot(s_ref[1].astype(jnp.bfloat16), t[:, 64:128],
                 preferred_element_type=jnp.float32)
    h2 = jnp.maximum(jnp.concatenate([a0, a1], axis=1) + self_ref[...], 0.0)
    o_ref[...] = jnp.maximum(
        jnp.dot(h2, dw_ref[...], preferred_element_type=jnp.float32), 0.0)


def _agg1(supports, q_other, q_self, dense_w, bm=512):
    return pl.pallas_call(
        _agg1_body,
        grid=(N // bm,),
        in_specs=[
            pl.BlockSpec((2, bm, N), lambda m: (0, m, 0)),
            pl.BlockSpec((N, 128), lambda m: (0, 0)),
            pl.BlockSpec((bm, 128), lambda m: (m, 1)),
            pl.BlockSpec((128, F_PAD), lambda m: (0, 0)),
        ],
        out_specs=pl.BlockSpec((bm, F_PAD), lambda m: (m, 0)),
        out_shape=jax.ShapeDtypeStruct((N, F_PAD), jnp.float32),
    )(supports, q_other, q_self, dense_w)


def _fin_body(rp_ref, wr_ref, wc_ref, o_ref):
    rp = rp_ref[:, :F_DIM]
    m0 = wc_ref[0, 0] * wr_ref[0:1, :] + wc_ref[1, 0] * wr_ref[1:2, :]
    m1 = wc_ref[0, 1] * wr_ref[0:1, :] + wc_ref[1, 1] * wr_ref[1:2, :]
    m = jnp.concatenate([m0, m1], axis=0)
    basis_t = lax.dot_general(m, rp, (((1,), (1,)), ((), ())),
                              preferred_element_type=jnp.float32)
    o_ref[...] = jnp.maximum(basis_t, 0.0)


def _finish(rp_rows, w_relation, weight_classifier, br=8192):
    out_t = pl.pallas_call(
        _fin_body,
        grid=(N_PAIRS // br,),
        in_specs=[
            pl.BlockSpec((br, F_PAD), lambda i: (i, 0)),
            pl.BlockSpec((2, F_DIM), lambda i: (0, 0)),
            pl.BlockSpec(memory_space=pltpu.SMEM),
        ],
        out_specs=pl.BlockSpec((2, br), lambda i: (0, i)),
        out_shape=jax.ShapeDtypeStruct((2, N_PAIRS), jnp.float32),
    )(rp_rows, w_relation, weight_classifier)
    return out_t.T


_NC, _NS = 2, 16
_NW = _NC * _NS
_CHUNK = 128
_CPW = N_PAIRS // _NW // _CHUNK
_PPW = N_PAIRS // _NW


def _sc_gather_mul(f_r, f_p, idx_r2, idx_p2):
    mesh = plsc.VectorSubcoreMesh(core_axis_name="c", subcore_axis_name="s",
                                  num_cores=_NC, num_subcores=_NS)

    @functools.partial(
        pl.kernel,
        out_type=jax.ShapeDtypeStruct((N_PAIRS, F_PAD), jnp.float32),
        mesh=mesh,
        scratch_types=[
            pltpu.VMEM((_CPW, _CHUNK), jnp.int32),
            pltpu.VMEM((_CPW, _CHUNK), jnp.int32),
            pltpu.VMEM((_CHUNK, F_PAD), jnp.float32),
            pltpu.VMEM((_CHUNK, F_PAD), jnp.float32),
            pltpu.VMEM((_CHUNK, F_PAD), jnp.float32),
            pltpu.VMEM((_CHUNK, F_PAD), jnp.float32),
            pltpu.VMEM((_CHUNK, F_PAD), jnp.float32),
            pltpu.SemaphoreType.DMA,
            pltpu.SemaphoreType.DMA,
            pltpu.SemaphoreType.DMA,
            pltpu.SemaphoreType.DMA,
        ],
    )
    def k(fr_hbm, fp_hbm, ir_hbm, ip_hbm, o_hbm,
          ir_v, ip_v, br0_v, br1_v, bp0_v, bp1_v, rp_v,
          sr0, sr1, sp0, sp1):
        wid = lax.axis_index("s") * _NC + lax.axis_index("c")
        pltpu.sync_copy(ir_hbm.at[pl.ds(wid * _CPW, _CPW)], ir_v)
        pltpu.sync_copy(ip_hbm.at[pl.ds(wid * _CPW, _CPW)], ip_v)
        rbufs, pbufs = (br0_v, br1_v), (bp0_v, bp1_v)
        rsems, psems = (sr0, sr1), (sp0, sp1)
        rcp = [None, None]
        pcp = [None, None]

        def _consume(slot, c):
            rcp[slot].wait()
            pcp[slot].wait()
            br, bp = rbufs[slot], pbufs[slot]

            def body(row, _):
                for g in range(F_PAD // 16):
                    sl = pl.ds(g * 16, 16)
                    rp_v[row, sl] = br[row, sl] * bp[row, sl]
                return _

            lax.fori_loop(0, _CHUNK, body, 0)
            off = wid * _PPW + c * _CHUNK
            pltpu.sync_copy(rp_v, o_hbm.at[pl.ds(off, _CHUNK)])

        for c in range(_CPW):
            slot = c % 2
            rcp[slot] = pltpu.async_copy(fr_hbm.at[ir_v.at[c]], rbufs[slot],
                                         rsems[slot])
            pcp[slot] = pltpu.async_copy(fp_hbm.at[ip_v.at[c]], pbufs[slot],
                                         psems[slot])
            if c > 0:
                _consume((c - 1) % 2, c - 1)
        _consume((_CPW - 1) % 2, _CPW - 1)

    return k(f_r, f_p, idx_r2, idx_p2)


def kernel(RNA_supports, protein_supports, RNA_inputs, protein_inputs,
           enc_w0, enc_w1, self_w0, self_w1,
           dense_w_rna, dense_w_prot, w_relation, weight_classifier,
           RNA_indices, protein_indices):
    wcat0 = jnp.concatenate([enc_w0[0], enc_w0[1], self_w0], axis=1)
    wcat1 = jnp.concatenate([enc_w1[0], enc_w1[1], self_w1], axis=1)

    pr = _proj(RNA_inputs, wcat0)
    pp = _proj(protein_inputs, wcat0)

    q_r = _agg0(RNA_supports, pp, pr, wcat1)
    q_p = _agg0(protein_supports, pr, pp, wcat1)

    dwr = jnp.pad(dense_w_rna, ((0, 0), (0, F_PAD - F_DIM)))
    dwp = jnp.pad(dense_w_prot, ((0, 0), (0, F_PAD - F_DIM)))
    f_r = _agg1(RNA_supports, q_p, q_r, dwr)
    f_p = _agg1(protein_supports, q_r, q_p, dwp)

    rp_rows = _sc_gather_mul(
        f_r, f_p,
        RNA_indices.reshape(_NW * _CPW, _CHUNK),
        protein_indices.reshape(_NW * _CPW, _CHUNK))
    return _finish(rp_rows, w_relation, weight_classifier)
